# Initial kernel scaffold; baseline (speedup 1.0000x reference)
import functools

import jax
import jax.numpy as jnp
from jax import lax
from jax.experimental import pallas as pl
from jax.experimental.pallas import tpu as pltpu
from jax.experimental.pallas import tpu_sc as plsc

N = 10000
NPAD = 10240
E = 320000
CHUNK = 128
NTILES = 32
NCH = 79
EPT = NCH * CHUNK
EPAD = EPT * NTILES
D_IN = 128
D_HID = 128
D_OUT = 16
BLK = 2048


def _sc_mesh():
    return plsc.VectorSubcoreMesh(core_axis_name="c", subcore_axis_name="s")


@functools.partial(
    pl.kernel,
    out_type=jax.ShapeDtypeStruct((2, NPAD, 1), jnp.float32),
    mesh=_sc_mesh(),
    scratch_types=[
        pltpu.VMEM((CHUNK,), jnp.int32),
        pltpu.VMEM((CHUNK, 1), jnp.float32),
        pltpu.VMEM_SHARED((NPAD, 1), jnp.float32),
    ],
)
def _deg_pass(dst_hbm, ones_hbm, degp_hbm, didx, ones_v, acc):
    c = lax.axis_index("c")
    s = lax.axis_index("s")
    wid = c * 16 + s
    rpt = NPAD // 16
    pltpu.sync_copy(ones_hbm.at[pl.ds(s * rpt, rpt)], acc.at[pl.ds(s * rpt, rpt)])
    pltpu.sync_copy(ones_hbm.at[pl.ds(0, CHUNK)], ones_v)
    plsc.subcore_barrier()

    base = wid * EPT

    def body(j, carry):
        off = base + j * CHUNK
        pltpu.sync_copy(dst_hbm.at[pl.ds(off, CHUNK)], didx)
        pltpu.sync_copy(ones_v, acc.at[didx], add=True)
        return carry

    lax.fori_loop(0, NCH, body, 0)
    plsc.subcore_barrier()
    pltpu.sync_copy(acc.at[pl.ds(s * rpt, rpt)], degp_hbm.at[c, pl.ds(s * rpt, rpt)])


def _make_agg(D):
    @functools.partial(
        pl.kernel,
        out_type=jax.ShapeDtypeStruct((2, NPAD, D), jnp.float32),
        mesh=_sc_mesh(),
        scratch_types=[
            pltpu.VMEM((CHUNK,), jnp.int32),
            pltpu.VMEM((CHUNK,), jnp.int32),
            pltpu.VMEM((CHUNK, D), jnp.float32),
            pltpu.VMEM_SHARED((NPAD, D), jnp.float32),
            pltpu.SemaphoreType.DMA,
        ],
    )
    def agg(hs_hbm, src_hbm, dst_hbm, out_hbm, sidx, didx, rows, acc, sem):
        c = lax.axis_index("c")
        s = lax.axis_index("s")
        wid = c * 16 + s
        rpt = NPAD // 16
        pltpu.sync_copy(hs_hbm.at[pl.ds(s * rpt, rpt)], acc.at[pl.ds(s * rpt, rpt)])
        plsc.subcore_barrier()

        base = wid * EPT

        def body(j, carry):
            off = base + j * CHUNK
            pltpu.sync_copy(src_hbm.at[pl.ds(off, CHUNK)], sidx)
            pltpu.sync_copy(dst_hbm.at[pl.ds(off, CHUNK)], didx)
            pltpu.async_copy(hs_hbm.at[sidx], rows, sem).wait()
            pltpu.sync_copy(rows, acc.at[didx], add=True)
            return carry

        lax.fori_loop(0, NCH, body, 0)
        plsc.subcore_barrier()
        pltpu.sync_copy(acc.at[pl.ds(s * rpt, rpt)], out_hbm.at[c, pl.ds(s * rpt, rpt)])

    return agg


_agg128 = _make_agg(D_HID)
_agg16 = _make_agg(D_OUT)


def _dis_body(degp_ref, dis_ref):
    deg = degp_ref[0] + degp_ref[1] - 1.0
    dis_ref[...] = lax.rsqrt(deg)


def _mm_body(x_ref, w_ref, o_ref):
    o_ref[...] = jnp.dot(x_ref[...], w_ref[...], preferred_element_type=jnp.float32)


def _scale_body(h_ref, dis_ref, o_ref):
    o_ref[...] = h_ref[...] * dis_ref[...]


def _mid_body(p_ref, hs_ref, dis_ref, b1_ref, w2_ref, o_ref):
    dis = dis_ref[...]
    agg = (p_ref[0] + p_ref[1] - hs_ref[...]) * dis + b1_ref[...]
    z = jnp.maximum(agg, 0.0)
    o_ref[...] = jnp.dot(z, w2_ref[...], preferred_element_type=jnp.float32) * dis


def _final_body(p_ref, hs_ref, dis_ref, b2_ref, o_ref):
    agg = (p_ref[0] + p_ref[1] - hs_ref[...]) * dis_ref[...] + b2_ref[...]
    o_ref[...] = jax.nn.sigmoid(agg)


def _row_spec(d):
    return pl.BlockSpec((BLK, d), lambda i: (i, 0))


def _full_spec(shape):
    nd = len(shape)
    return pl.BlockSpec(shape, lambda i: (0,) * nd)


def _part_spec(d):
    return pl.BlockSpec((2, BLK, d), lambda i: (0, i, 0))


_GRID = NPAD // BLK


def _tc(body, out_d, in_specs):
    return pl.pallas_call(
        body,
        grid=(_GRID,),
        in_specs=in_specs,
        out_specs=_row_spec(out_d),
        out_shape=jax.ShapeDtypeStruct((NPAD, out_d), jnp.float32),
    )


def kernel(x, edge_index, W1, b1, W2, b2):
    ei = edge_index.astype(jnp.int32)
    src = jnp.concatenate([ei[0], jnp.zeros((EPAD - E,), jnp.int32)])
    dst = jnp.concatenate([ei[1], jnp.full((EPAD - E,), N, jnp.int32)])
    x_p = jnp.zeros((NPAD, D_IN), jnp.float32).at[:N].set(x)
    ones_col = jnp.ones((NPAD, 1), jnp.float32)
    b1_r = b1.reshape(1, D_HID)
    b2_r = b2.reshape(1, D_OUT)

    degp = _deg_pass(dst, ones_col)

    dis = pl.pallas_call(
        _dis_body,
        in_specs=[pl.BlockSpec((2, NPAD, 1), lambda: (0, 0, 0))],
        out_specs=pl.BlockSpec((NPAD, 1), lambda: (0, 0)),
        out_shape=jax.ShapeDtypeStruct((NPAD, 1), jnp.float32),
    )(degp)

    h1 = _tc(_mm_body, D_HID, [_row_spec(D_IN), _full_spec((D_IN, D_HID))])(x_p, W1)
    hs1 = _tc(_scale_body, D_HID, [_row_spec(D_HID), _row_spec(1)])(h1, dis)

    aggp1 = _agg128(hs1, src, dst)

    hs2 = _tc(
        _mid_body, D_OUT,
        [_part_spec(D_HID), _row_spec(D_HID), _row_spec(1),
         _full_spec((1, D_HID)), _full_spec((D_HID, D_OUT))],
    )(aggp1, hs1, dis, b1_r, W2)

    aggp2 = _agg16(hs2, src, dst)

    out_p = _tc(
        _final_body, D_OUT,
        [_part_spec(D_OUT), _row_spec(D_OUT), _row_spec(1), _full_spec((1, D_OUT))],
    )(aggp2, hs2, dis, b2_r)

    return out_p[:N]


# speedup vs baseline: 14.6581x; 14.6581x over previous
import functools

import jax
import jax.numpy as jnp
from jax import lax
from jax.experimental import pallas as pl
from jax.experimental.pallas import tpu as pltpu
from jax.experimental.pallas import tpu_sc as plsc

N = 10000
NPAD = 10240
E = 320000
CHUNK = 128
NTILES = 32
NCH = 79
EPT = NCH * CHUNK
EPAD = EPT * NTILES
D_IN = 128
D_HID = 128
D_OUT = 16
BLK = 2048


def _sc_mesh():
    return plsc.VectorSubcoreMesh(core_axis_name="c", subcore_axis_name="s")


_SC_PARAMS = pltpu.CompilerParams(use_tc_tiling_on_sc=False)


@functools.partial(
    pl.kernel,
    out_type=jax.ShapeDtypeStruct((2, NPAD, 1), jnp.float32),
    mesh=_sc_mesh(),
    scratch_types=[
        pltpu.VMEM((CHUNK,), jnp.int32),
        pltpu.VMEM((CHUNK, 1), jnp.float32),
        pltpu.VMEM_SHARED((NPAD, 1), jnp.float32),
    ],
    compiler_params=_SC_PARAMS,
)
def _deg_pass(dst_hbm, ones_hbm, degp_hbm, didx, ones_v, acc):
    c = lax.axis_index("c")
    s = lax.axis_index("s")
    wid = c * 16 + s
    rpt = NPAD // 16
    pltpu.sync_copy(ones_hbm.at[pl.ds(s * rpt, rpt)], acc.at[pl.ds(s * rpt, rpt)])
    pltpu.sync_copy(ones_hbm.at[pl.ds(0, CHUNK)], ones_v)
    plsc.subcore_barrier()

    base = wid * EPT

    def body(j, carry):
        off = base + j * CHUNK
        pltpu.sync_copy(dst_hbm.at[pl.ds(off, CHUNK)], didx)
        pltpu.sync_copy(ones_v, acc.at[didx], add=True)
        return carry

    lax.fori_loop(0, NCH, body, 0)
    plsc.subcore_barrier()
    pltpu.sync_copy(acc.at[pl.ds(s * rpt, rpt)], degp_hbm.at[c, pl.ds(s * rpt, rpt)])


def _make_agg(D):
    @functools.partial(
        pl.kernel,
        out_type=jax.ShapeDtypeStruct((2, NPAD, D), jnp.float32),
        mesh=_sc_mesh(),
        scratch_types=[
            pltpu.VMEM((CHUNK,), jnp.int32),
            pltpu.VMEM((CHUNK,), jnp.int32),
            pltpu.VMEM((CHUNK, D), jnp.float32),
            pltpu.VMEM_SHARED((NPAD, D), jnp.float32),
            pltpu.SemaphoreType.DMA,
        ],
        compiler_params=_SC_PARAMS,
    )
    def agg(hs_hbm, src_hbm, dst_hbm, out_hbm, sidx, didx, rows, acc, sem):
        c = lax.axis_index("c")
        s = lax.axis_index("s")
        wid = c * 16 + s
        rpt = NPAD // 16
        pltpu.sync_copy(hs_hbm.at[pl.ds(s * rpt, rpt)], acc.at[pl.ds(s * rpt, rpt)])
        plsc.subcore_barrier()

        base = wid * EPT

        def body(j, carry):
            off = base + j * CHUNK
            pltpu.sync_copy(src_hbm.at[pl.ds(off, CHUNK)], sidx)
            pltpu.sync_copy(dst_hbm.at[pl.ds(off, CHUNK)], didx)
            pltpu.async_copy(hs_hbm.at[sidx], rows, sem).wait()
            pltpu.sync_copy(rows, acc.at[didx], add=True)
            return carry

        lax.fori_loop(0, NCH, body, 0)
        plsc.subcore_barrier()
        pltpu.sync_copy(acc.at[pl.ds(s * rpt, rpt)], out_hbm.at[c, pl.ds(s * rpt, rpt)])

    return agg


_agg128 = _make_agg(D_HID)
_agg16 = _make_agg(D_OUT)


def _dis_body(degp_ref, dis_ref):
    deg = degp_ref[0] + degp_ref[1] - 1.0
    dis_ref[...] = lax.rsqrt(deg)


def _mm_body(x_ref, w_ref, o_ref):
    o_ref[...] = jnp.dot(x_ref[...], w_ref[...], preferred_element_type=jnp.float32)


def _scale_body(h_ref, dis_ref, o_ref):
    o_ref[...] = h_ref[...] * dis_ref[...]


def _mid_body(p_ref, hs_ref, dis_ref, b1_ref, w2_ref, o_ref):
    dis = dis_ref[...]
    agg = (p_ref[0] + p_ref[1] - hs_ref[...]) * dis + b1_ref[...]
    z = jnp.maximum(agg, 0.0)
    o_ref[...] = jnp.dot(z, w2_ref[...], preferred_element_type=jnp.float32) * dis


def _final_body(p_ref, hs_ref, dis_ref, b2_ref, o_ref):
    agg = (p_ref[0] + p_ref[1] - hs_ref[...]) * dis_ref[...] + b2_ref[...]
    o_ref[...] = jax.nn.sigmoid(agg)


def _row_spec(d):
    return pl.BlockSpec((BLK, d), lambda i: (i, 0))


def _full_spec(shape):
    nd = len(shape)
    return pl.BlockSpec(shape, lambda i: (0,) * nd)


def _part_spec(d):
    return pl.BlockSpec((2, BLK, d), lambda i: (0, i, 0))


_GRID = NPAD // BLK


def _tc(body, out_d, in_specs):
    return pl.pallas_call(
        body,
        grid=(_GRID,),
        in_specs=in_specs,
        out_specs=_row_spec(out_d),
        out_shape=jax.ShapeDtypeStruct((NPAD, out_d), jnp.float32),
    )


def kernel(x, edge_index, W1, b1, W2, b2):
    ei = edge_index.astype(jnp.int32)
    src = jnp.concatenate([ei[0], jnp.zeros((EPAD - E,), jnp.int32)])
    dst = jnp.concatenate([ei[1], jnp.full((EPAD - E,), N, jnp.int32)])
    x_p = jnp.zeros((NPAD, D_IN), jnp.float32).at[:N].set(x)
    ones_col = jnp.ones((NPAD, 1), jnp.float32)
    b1_r = b1.reshape(1, D_HID)
    b2_r = b2.reshape(1, D_OUT)

    degp = _deg_pass(dst, ones_col)

    dis = pl.pallas_call(
        _dis_body,
        in_specs=[pl.BlockSpec((2, NPAD, 1), lambda: (0, 0, 0))],
        out_specs=pl.BlockSpec((NPAD, 1), lambda: (0, 0)),
        out_shape=jax.ShapeDtypeStruct((NPAD, 1), jnp.float32),
    )(degp)

    h1 = _tc(_mm_body, D_HID, [_row_spec(D_IN), _full_spec((D_IN, D_HID))])(x_p, W1)
    hs1 = _tc(_scale_body, D_HID, [_row_spec(D_HID), _row_spec(1)])(h1, dis)

    aggp1 = _agg128(hs1, src, dst)

    hs2 = _tc(
        _mid_body, D_OUT,
        [_part_spec(D_HID), _row_spec(D_HID), _row_spec(1),
         _full_spec((1, D_HID)), _full_spec((D_HID, D_OUT))],
    )(aggp1, hs1, dis, b1_r, W2)

    aggp2 = _agg16(hs2, src, dst)

    out_p = _tc(
        _final_body, D_OUT,
        [_part_spec(D_OUT), _row_spec(D_OUT), _row_spec(1), _full_spec((1, D_OUT))],
    )(aggp2, hs2, dis, b2_r)

    return out_p[:N]


# speedup vs baseline: 15.9840x; 1.0905x over previous
import functools

import jax
import jax.numpy as jnp
from jax import lax
from jax.experimental import pallas as pl
from jax.experimental.pallas import tpu as pltpu
from jax.experimental.pallas import tpu_sc as plsc

N = 10000
NPAD = 10240
E = 320000
CHUNK = 128
NTILES = 32
NCH = 80
EPT = NCH * CHUNK
EPAD = EPT * NTILES
NROWS = EPAD // CHUNK
D_IN = 128
D_HID = 128
D_OUT = 16
BLK = 2048
NBUF = 2
DEG_GRP = 8


def _sc_mesh():
    return plsc.VectorSubcoreMesh(core_axis_name="c", subcore_axis_name="s")


_SC_PARAMS = pltpu.CompilerParams(use_tc_tiling_on_sc=False)


@functools.partial(
    pl.kernel,
    out_type=jax.ShapeDtypeStruct((2, NPAD, 1), jnp.float32),
    mesh=_sc_mesh(),
    scratch_types=[
        pltpu.VMEM((NCH, CHUNK), jnp.int32),
        pltpu.VMEM((CHUNK, 1), jnp.float32),
        pltpu.VMEM_SHARED((NPAD, 1), jnp.float32),
        pltpu.SemaphoreType.DMA,
    ],
    compiler_params=_SC_PARAMS,
)
def _deg_pass(dst_hbm, ones_hbm, degp_hbm, didx, ones_v, acc, ssem):
    c = lax.axis_index("c")
    s = lax.axis_index("s")
    wid = c * 16 + s
    rpt = NPAD // 16
    pltpu.sync_copy(ones_hbm.at[pl.ds(s * rpt, rpt)], acc.at[pl.ds(s * rpt, rpt)])
    pltpu.sync_copy(ones_hbm.at[pl.ds(0, CHUNK)], ones_v)
    pltpu.sync_copy(dst_hbm.at[pl.ds(wid * NCH, NCH)], didx)
    plsc.subcore_barrier()

    def body(i, carry):
        descs = []
        for b in range(DEG_GRP):
            j = i * DEG_GRP + b
            descs.append(pltpu.async_copy(ones_v, acc.at[didx.at[j]], ssem, add=True))
        for d in descs:
            d.wait()
        return carry

    lax.fori_loop(0, NCH // DEG_GRP, body, 0)
    plsc.subcore_barrier()
    pltpu.sync_copy(acc.at[pl.ds(s * rpt, rpt)], degp_hbm.at[c, pl.ds(s * rpt, rpt)])


IDXR = 4
UNROLL = 4


def _make_agg(D):
    @functools.partial(
        pl.kernel,
        out_type=jax.ShapeDtypeStruct((2, NPAD, D), jnp.float32),
        mesh=_sc_mesh(),
        scratch_types=[
            pltpu.VMEM((NCH, CHUNK), jnp.int32),
            [pltpu.VMEM((CHUNK,), jnp.int32)] * IDXR,
            [pltpu.VMEM((CHUNK, D), jnp.float32)] * NBUF,
            pltpu.VMEM_SHARED((NPAD, D), jnp.float32),
            [pltpu.SemaphoreType.DMA] * IDXR,
            [pltpu.SemaphoreType.DMA] * NBUF,
        ],
        compiler_params=_SC_PARAMS,
    )
    def agg(hs_hbm, src_hbm, dst_hbm, out_hbm, sidx, didxr, bufs, acc, isems, gsems):
        c = lax.axis_index("c")
        s = lax.axis_index("s")
        wid = c * 16 + s
        rpt = NPAD // 16
        pltpu.sync_copy(hs_hbm.at[pl.ds(s * rpt, rpt)], acc.at[pl.ds(s * rpt, rpt)])
        pltpu.sync_copy(src_hbm.at[pl.ds(wid * NCH, NCH)], sidx)
        row0 = wid * NCH
        for u in range(IDXR):
            pltpu.async_copy(dst_hbm.at[row0 + u], didxr[u], isems[u])
        pltpu.async_copy(hs_hbm.at[sidx.at[0]], bufs[0], gsems[0])
        plsc.subcore_barrier()

        def body(i, carry):
            for u in range(UNROLL):
                j = i * UNROLL + u
                nxt = jnp.minimum(j + 1, NCH - 1)
                pltpu.async_copy(hs_hbm.at[sidx.at[nxt]], bufs[(u + 1) % NBUF],
                                 gsems[(u + 1) % NBUF])
                pltpu.make_async_copy(hs_hbm.at[sidx.at[j]], bufs[u % NBUF],
                                      gsems[u % NBUF]).wait()
                pltpu.make_async_copy(dst_hbm.at[row0], didxr[u % IDXR],
                                      isems[u % IDXR]).wait()
                pltpu.sync_copy(bufs[u % NBUF], acc.at[didxr[u % IDXR]], add=True)
                nxi = jnp.minimum(j + IDXR, NCH - 1)
                pltpu.async_copy(dst_hbm.at[row0 + nxi], didxr[u % IDXR],
                                 isems[u % IDXR])
            return carry

        lax.fori_loop(0, NCH // UNROLL, body, 0)
        pltpu.make_async_copy(hs_hbm.at[sidx.at[NCH - 1]], bufs[0], gsems[0]).wait()
        for u in range(IDXR):
            pltpu.make_async_copy(dst_hbm.at[row0], didxr[u], isems[u]).wait()
        plsc.subcore_barrier()
        pltpu.sync_copy(acc.at[pl.ds(s * rpt, rpt)], out_hbm.at[c, pl.ds(s * rpt, rpt)])

    return agg


_agg128 = _make_agg(D_HID)
_agg16 = _make_agg(D_OUT)


def _dis_body(degp_ref, dis_ref):
    deg = degp_ref[0] + degp_ref[1] - 1.0
    dis_ref[...] = lax.rsqrt(deg)


def _mm_body(x_ref, w_ref, o_ref):
    o_ref[...] = jnp.dot(x_ref[...], w_ref[...], preferred_element_type=jnp.float32)


def _scale_body(h_ref, dis_ref, o_ref):
    o_ref[...] = h_ref[...] * dis_ref[...]


def _mid_body(p_ref, hs_ref, dis_ref, b1_ref, w2_ref, o_ref):
    dis = dis_ref[...]
    agg = (p_ref[0] + p_ref[1] - hs_ref[...]) * dis + b1_ref[...]
    z = jnp.maximum(agg, 0.0)
    o_ref[...] = jnp.dot(z, w2_ref[...], preferred_element_type=jnp.float32) * dis


def _final_body(p_ref, hs_ref, dis_ref, b2_ref, o_ref):
    agg = (p_ref[0] + p_ref[1] - hs_ref[...]) * dis_ref[...] + b2_ref[...]
    o_ref[...] = jax.nn.sigmoid(agg)


def _row_spec(d):
    return pl.BlockSpec((BLK, d), lambda i: (i, 0))


def _full_spec(shape):
    nd = len(shape)
    return pl.BlockSpec(shape, lambda i: (0,) * nd)


def _part_spec(d):
    return pl.BlockSpec((2, BLK, d), lambda i: (0, i, 0))


_GRID = NPAD // BLK


def _tc(body, out_d, in_specs):
    return pl.pallas_call(
        body,
        grid=(_GRID,),
        in_specs=in_specs,
        out_specs=_row_spec(out_d),
        out_shape=jax.ShapeDtypeStruct((NPAD, out_d), jnp.float32),
    )


def kernel(x, edge_index, W1, b1, W2, b2):
    ei = edge_index.astype(jnp.int32)
    src = jnp.concatenate([ei[0], jnp.zeros((EPAD - E,), jnp.int32)])
    dst = jnp.concatenate([ei[1], jnp.full((EPAD - E,), N, jnp.int32)])
    src2 = src.reshape(NROWS, CHUNK)
    dst2 = dst.reshape(NROWS, CHUNK)
    x_p = jnp.zeros((NPAD, D_IN), jnp.float32).at[:N].set(x)
    ones_col = jnp.ones((NPAD, 1), jnp.float32)
    b1_r = b1.reshape(1, D_HID)
    b2_r = b2.reshape(1, D_OUT)

    degp = _deg_pass(dst2, ones_col)

    dis = pl.pallas_call(
        _dis_body,
        in_specs=[pl.BlockSpec((2, NPAD, 1), lambda: (0, 0, 0))],
        out_specs=pl.BlockSpec((NPAD, 1), lambda: (0, 0)),
        out_shape=jax.ShapeDtypeStruct((NPAD, 1), jnp.float32),
    )(degp)

    h1 = _tc(_mm_body, D_HID, [_row_spec(D_IN), _full_spec((D_IN, D_HID))])(x_p, W1)
    hs1 = _tc(_scale_body, D_HID, [_row_spec(D_HID), _row_spec(1)])(h1, dis)

    aggp1 = _agg128(hs1, src2, dst2)

    hs2 = _tc(
        _mid_body, D_OUT,
        [_part_spec(D_HID), _row_spec(D_HID), _row_spec(1),
         _full_spec((1, D_HID)), _full_spec((D_HID, D_OUT))],
    )(aggp1, hs1, dis, b1_r, W2)

    aggp2 = _agg16(hs2, src2, dst2)

    out_p = _tc(
        _final_body, D_OUT,
        [_part_spec(D_OUT), _row_spec(D_OUT), _row_spec(1), _full_spec((1, D_OUT))],
    )(aggp2, hs2, dis, b2_r)

    return out_p[:N]
